# baseline (speedup 1.0000x reference)
import math

import jax
import jax.numpy as jnp
from jax import lax
from jax.experimental import pallas as pl
from jax.experimental.pallas import tpu as pltpu
from jax.experimental.pallas import tpu_sc as plsc

N_TOK = 32768
D = 128
N_BATCH = 16
N_SEG = 1024
NC = 2
NS = 16
NW = NC * NS
ROWS_PER_TILE = N_SEG // NW
TOK_PER_TILE = N_TOK // NW

R_MAT = 256
K4_SUB = 128

TQ = 4
K2_BLK = 1024
K2_ROUNDS = N_TOK // (TQ * K2_BLK)
CNT_PAD = N_SEG + 16

_SC_PARAMS = pltpu.CompilerParams(needs_layout_passes=False)

K1_BLOCKS = 8
K1_R = R_MAT // K1_BLOCKS
K1_TOK = N_TOK // K1_BLOCKS


def _k1a_body(feats3_ref, lid_ref, cu_ref, attw_ref, attb_ref, fccB_ref,
              fccb_ref, featsF_ref, logits_ref, rowid_ref):
    g = pl.program_id(0)
    f3 = feats3_ref[...]
    aw = attw_ref[...].reshape(1, 1, D)
    logits_ref[...] = jnp.sum(f3 * aw, axis=2) + attb_ref[0]

    i0 = lax.broadcasted_iota(jnp.int32, (K1_R, D), 0)
    i1 = lax.broadcasted_iota(jnp.int32, (K1_R, D), 1)
    idx = g * K1_TOK + i0 * D + i1
    b = jnp.zeros((K1_R, D), jnp.int32)
    for j in range(1, N_BATCH):
        b = b + (idx >= cu_ref[j]).astype(jnp.int32)

    lid = lid_ref[...]
    rowid_ref[...] = (lid & 31) * 32 + b * 2 + (lid >> 5)

    feats = f3.reshape(K1_TOK, D)
    featsF_ref[...] = (
        lax.dot_general(feats, fccB_ref[...], (((1,), (1,)), ((), ())),
                        preferred_element_type=jnp.float32)
        + fccb_ref[...]
    )


def _k1b_body(logits_ref, cu_ref, w_ref):
    logits = logits_ref[...]
    i0 = lax.broadcasted_iota(jnp.int32, (R_MAT, D), 0)
    i1 = lax.broadcasted_iota(jnp.int32, (R_MAT, D), 1)
    idx = i0 * D + i1
    b = jnp.zeros((R_MAT, D), jnp.int32)
    for j in range(1, N_BATCH):
        b = b + (idx >= cu_ref[j]).astype(jnp.int32)

    m = jnp.max(logits)
    e = jnp.exp(logits - m)
    denom = jnp.ones((R_MAT, D), jnp.float32)
    for j in range(N_BATCH):
        mask = b == j
        zj = jnp.sum(jnp.where(mask, e, 0.0))
        denom = jnp.where(mask, zj, denom)
    w_ref[...] = e / denom


def _run_k1(feats, lid_mat, cu_seqlens, attn_w, attn_b, fccB, fcc_b):
    feats3 = feats.reshape(R_MAT, D, D)
    featsF, logits, rowid = pl.pallas_call(
        _k1a_body,
        grid=(K1_BLOCKS,),
        out_shape=[
            jax.ShapeDtypeStruct((N_TOK, D), jnp.float32),
            jax.ShapeDtypeStruct((R_MAT, D), jnp.float32),
            jax.ShapeDtypeStruct((R_MAT, D), jnp.int32),
        ],
        in_specs=[
            pl.BlockSpec((K1_R, D, D), lambda i: (i, 0, 0)),
            pl.BlockSpec((K1_R, D), lambda i: (i, 0)),
            pl.BlockSpec(memory_space=pltpu.SMEM),
            pl.BlockSpec((1, D), lambda i: (0, 0)),
            pl.BlockSpec(memory_space=pltpu.SMEM),
            pl.BlockSpec((D, D), lambda i: (0, 0)),
            pl.BlockSpec((1, D), lambda i: (0, 0)),
        ],
        out_specs=[
            pl.BlockSpec((K1_TOK, D), lambda i: (i, 0)),
            pl.BlockSpec((K1_R, D), lambda i: (i, 0)),
            pl.BlockSpec((K1_R, D), lambda i: (i, 0)),
        ],
    )(feats3, lid_mat, cu_seqlens, attn_w, attn_b, fccB, fcc_b)

    w = pl.pallas_call(
        _k1b_body,
        out_shape=jax.ShapeDtypeStruct((R_MAT, D), jnp.float32),
        in_specs=[
            pl.BlockSpec(memory_space=pltpu.VMEM),
            pl.BlockSpec(memory_space=pltpu.SMEM),
        ],
    )(logits, cu_seqlens)
    return featsF, w, rowid


def _k2_body(featsG_hbm, rid_hbm, w_hbm, avg_hbm, max_hbm, att_hbm,
             stage, stage_cnt,
             vals0, vals1, ridb0, ridb1, wb0, wb1,
             acc_s, acc_m, acc_a, cntb, mb_s,
             dsem0, dsem1):
    c = lax.axis_index("c")
    s = lax.axis_index("s")
    q = lax.shift_right_logical(s, 2)
    g = s & 3
    dg = c * 4 + g

    vals = (vals0, vals1)
    ridbs = (ridb0, ridb1)
    wbs = (wb0, wb1)
    dsems = (dsem0, dsem1)

    zeros16 = jnp.zeros((16,), jnp.float32)
    neglarge = jnp.full((16,), -3.4e38, jnp.float32)
    one_first = (lax.iota(jnp.int32, 16) == 0).astype(jnp.float32)

    def init_fn(r, _):
        for k in range(8):
            sl = pl.ds(k * 16, 16)
            acc_s[r, sl] = zeros16
            acc_m[r, sl] = neglarge
            acc_a[r, sl] = zeros16
        return 0

    lax.fori_loop(0, N_SEG // 8, init_fn, 0)

    def cinit_fn(r, _):
        cntb[pl.ds(r * 16, 16)] = zeros16
        return 0

    lax.fori_loop(0, CNT_PAD // 16, cinit_fn, 0)

    def srcs(r):
        t0 = pl.multiple_of(q * (N_TOK // TQ) + r * K2_BLK, K2_BLK)
        t08 = pl.multiple_of(
            q * (N_TOK // TQ // 8) + r * (K2_BLK // 8), K2_BLK // 8)
        return (featsG_hbm.at[dg, pl.ds(t08, K2_BLK // 8), :],
                rid_hbm.at[pl.ds(t0, K2_BLK)],
                w_hbm.at[pl.ds(t0, K2_BLK)])

    def issue(r):
        p = r & 1
        fs, rs, ws = srcs(r)
        pltpu.async_copy(fs, vals[p], dsems[p])
        pltpu.async_copy(rs, ridbs[p], dsems[p])
        pltpu.async_copy(ws, wbs[p], dsems[p])

    def wait(r):
        p = r & 1
        fs, rs, ws = srcs(r)
        pltpu.make_async_copy(fs, vals[p], dsems[p]).wait()
        pltpu.make_async_copy(rs, ridbs[p], dsems[p]).wait()
        pltpu.make_async_copy(ws, wbs[p], dsems[p]).wait()

    issue(0)
    for r in range(K2_ROUNDS):
        p = r & 1
        if r + 1 < K2_ROUNDS:
            issue(r + 1)
        wait(r)
        vbuf = vals[p]
        ridb = ridbs[p]
        wb = wbs[p]

        def acc16(v, _, vbuf=vbuf, ridb=ridb, wb=wb):
            rv = ridb[pl.ds(v * 16, 16)]
            wv = wb[pl.ds(v * 16, 16)]
            for l in range(16):
                seg = rv[l]
                wt = wv[l]
                sr = lax.shift_right_logical(seg, 3)
                so = pl.ds((seg & 7) * 16, 16)
                val = vbuf[2 * v + (l >> 3), pl.ds((l & 7) * 16, 16)]
                acc_s[sr, so] = acc_s[sr, so] + val
                acc_m[sr, so] = jnp.maximum(acc_m[sr, so], val)
                acc_a[sr, so] = acc_a[sr, so] + val * wt
            return 0

        lax.fori_loop(0, K2_BLK // 16, acc16, 0)

        def cnt16(v, _, ridb=ridb):
            rv = ridb[pl.ds(g * (K2_BLK // 4) + v * 16, 16)]
            for l in range(16):
                seg = rv[l]
                cw = cntb[pl.ds(seg, 16)]
                cntb[pl.ds(seg, 16)] = cw + one_first
            return 0

        lax.fori_loop(0, K2_BLK // 4 // 16, cnt16, 0)

    SEGB = N_SEG // TQ
    seg0 = pl.multiple_of(q * SEGB, SEGB)
    row0 = pl.multiple_of(q * (SEGB // 8), SEGB // 8)
    ms = acc_s.at[pl.ds(0, SEGB // 8)]
    mm = acc_m.at[pl.ds(0, SEGB // 8)]
    ma = acc_a.at[pl.ds(0, SEGB // 8)]

    pltpu.sync_copy(cntb, stage_cnt.at[s])

    for t, (accX, mX) in enumerate(((acc_s, ms), (acc_m, mm), (acc_a, ma))):
        pltpu.sync_copy(accX, stage.at[s])
        plsc.subcore_barrier()

        for q2 in range(TQ):
            s2 = q2 * 4 + g
            pltpu.sync_copy(stage.at[s2, pl.ds(row0, SEGB // 8)], mb_s)

            def mrg_fn(r2, _, q2=q2, t=t, mX=mX):
                for k in range(8):
                    sl = pl.ds(k * 16, 16)
                    if q2 == 0:
                        mX[r2, sl] = mb_s[r2, sl]
                    elif t == 1:
                        mX[r2, sl] = jnp.maximum(mX[r2, sl], mb_s[r2, sl])
                    else:
                        mX[r2, sl] = mX[r2, sl] + mb_s[r2, sl]
                return 0

            lax.fori_loop(0, SEGB // 8, mrg_fn, 0)
        plsc.subcore_barrier()

    for s2 in range(16):
        pltpu.sync_copy(stage_cnt.at[s2, pl.ds(seg0, SEGB)],
                        wb0.at[pl.ds(0, SEGB)])

        def cmrg_fn(v, _, s2=s2):
            sl = pl.ds(v * 16, 16)
            if s2 == 0:
                cntb[sl] = wb0[sl]
            else:
                cntb[sl] = cntb[sl] + wb0[sl]
            return 0

        lax.fori_loop(0, SEGB // 16, cmrg_fn, 0)

    def fin_fn(sl_i, _):
        r2 = lax.shift_right_logical(sl_i, 3)
        so = pl.ds((sl_i & 7) * 16, 16)
        cf = cntb[pl.ds(sl_i, 16)][0]
        has = cf > 0.0
        ms[r2, so] = ms[r2, so] / jnp.maximum(cf, 1.0)
        mm[r2, so] = jnp.where(has, mm[r2, so], zeros16)
        return 0

    lax.fori_loop(0, SEGB, fin_fn, 0)

    pltpu.sync_copy(ms, avg_hbm.at[dg, pl.ds(row0, SEGB // 8), :])
    pltpu.sync_copy(mm, max_hbm.at[dg, pl.ds(row0, SEGB // 8), :])
    pltpu.sync_copy(ma, att_hbm.at[dg, pl.ds(row0, SEGB // 8), :])


def _run_k2(featsG, rowid_flat, w_flat):
    mesh = plsc.VectorSubcoreMesh(core_axis_name="c", subcore_axis_name="s")
    f = pl.kernel(
        _k2_body,
        out_type=[
            jax.ShapeDtypeStruct((8, N_SEG // 8, D), jnp.float32),
            jax.ShapeDtypeStruct((8, N_SEG // 8, D), jnp.float32),
            jax.ShapeDtypeStruct((8, N_SEG // 8, D), jnp.float32),
        ],
        mesh=mesh,
        compiler_params=_SC_PARAMS,
        scratch_types=[
            pltpu.VMEM_SHARED((16, N_SEG // 8, D), jnp.float32),
            pltpu.VMEM_SHARED((16, CNT_PAD), jnp.float32),
            pltpu.VMEM((K2_BLK // 8, D), jnp.float32),
            pltpu.VMEM((K2_BLK // 8, D), jnp.float32),
            pltpu.VMEM((K2_BLK,), jnp.int32),
            pltpu.VMEM((K2_BLK,), jnp.int32),
            pltpu.VMEM((K2_BLK,), jnp.float32),
            pltpu.VMEM((K2_BLK,), jnp.float32),
            pltpu.VMEM((N_SEG // 8, D), jnp.float32),
            pltpu.VMEM((N_SEG // 8, D), jnp.float32),
            pltpu.VMEM((N_SEG // 8, D), jnp.float32),
            pltpu.VMEM((CNT_PAD,), jnp.float32),
            pltpu.VMEM((N_SEG // TQ // 8, D), jnp.float32),
            pltpu.SemaphoreType.DMA,
            pltpu.SemaphoreType.DMA,
        ],
    )
    return f(featsG, rowid_flat, w_flat)


def _k3_body(avg_ref, max_ref, att_ref, w1_ref, b1_ref, w2_ref, b2_ref,
             fccA_ref, nl2_ref):
    fcat = jnp.concatenate([avg_ref[...], max_ref[...], att_ref[...]], axis=1)
    h = lax.dot_general(fcat, w1_ref[...], (((1,), (1,)), ((), ())),
                        preferred_element_type=jnp.float32) + b1_ref[...]
    h = 0.5 * h * (1.0 + lax.erf(h * (1.0 / math.sqrt(2.0))))
    nl = lax.dot_general(h, w2_ref[...], (((1,), (1,)), ((), ())),
                         preferred_element_type=jnp.float32) + b2_ref[...]
    nl2_ref[...] = lax.dot_general(nl, fccA_ref[...], (((1,), (1,)), ((), ())),
                                   preferred_element_type=jnp.float32)


def _run_k3(avgP, maxP, attP, fc1_w, fc1_b, fc2_w, fc2_b, fccA):
    return pl.pallas_call(
        _k3_body,
        out_shape=jax.ShapeDtypeStruct((N_SEG, D), jnp.float32),
    )(avgP, maxP, attP, fc1_w, fc1_b.reshape(1, -1), fc2_w,
      fc2_b.reshape(1, -1), fccA)


def _k4_body(featsF_hbm, nl2_hbm, rid_hbm, out_hbm,
             ridx0, ridx1, rows0, rows1, fbuf0, fbuf1,
             sem0, sem1, osem0, osem1):
    wid = lax.axis_index("s") * NC + lax.axis_index("c")
    base = wid * TOK_PER_TILE
    n_sub = TOK_PER_TILE // K4_SUB

    ridx = (ridx0, ridx1)
    rows = (rows0, rows1)
    fbuf = (fbuf0, fbuf1)
    sems = (sem0, sem1)
    osems = (osem0, osem1)

    def issue(s):
        p = s & 1
        t0 = base + s * K4_SUB
        pltpu.sync_copy(rid_hbm.at[pl.ds(t0, K4_SUB)], ridx[p])
        pltpu.async_copy(nl2_hbm.at[ridx[p]], rows[p], sems[p])
        pltpu.async_copy(featsF_hbm.at[pl.ds(t0, K4_SUB), :], fbuf[p], sems[p])

    def wait_in(s):
        p = s & 1
        t0 = base + s * K4_SUB
        pltpu.make_async_copy(nl2_hbm.at[ridx[p]], rows[p], sems[p]).wait()
        pltpu.make_async_copy(featsF_hbm.at[pl.ds(t0, K4_SUB), :], fbuf[p],
                              sems[p]).wait()

    def wait_out(s):
        p = s & 1
        t0 = base + s * K4_SUB
        pltpu.make_async_copy(fbuf[p], out_hbm.at[pl.ds(t0, K4_SUB), :],
                              osems[p]).wait()

    issue(0)
    for s in range(n_sub):
        p = s & 1
        t0 = base + s * K4_SUB
        if s + 1 < n_sub:
            if s >= 1:
                wait_out(s - 1)
            issue(s + 1)
        wait_in(s)

        def row_fn(r, _, p=p):
            for d in range(D // 16):
                sl = pl.ds(d * 16, 16)
                fbuf[p][r, sl] = fbuf[p][r, sl] + rows[p][r, sl]
            return 0

        lax.fori_loop(0, K4_SUB, row_fn, 0)
        pltpu.async_copy(fbuf[p], out_hbm.at[pl.ds(t0, K4_SUB), :], osems[p])
    if n_sub >= 2:
        wait_out(n_sub - 2)
    wait_out(n_sub - 1)


def _run_k4(featsF, nl2, rowid_flat):
    mesh = plsc.VectorSubcoreMesh(core_axis_name="c", subcore_axis_name="s")
    f = pl.kernel(
        _k4_body,
        out_type=jax.ShapeDtypeStruct((N_TOK, D), jnp.float32),
        mesh=mesh,
        compiler_params=_SC_PARAMS,
        scratch_types=[
            pltpu.VMEM((K4_SUB,), jnp.int32),
            pltpu.VMEM((K4_SUB,), jnp.int32),
            pltpu.VMEM((K4_SUB, D), jnp.float32),
            pltpu.VMEM((K4_SUB, D), jnp.float32),
            pltpu.VMEM((K4_SUB, D), jnp.float32),
            pltpu.VMEM((K4_SUB, D), jnp.float32),
            pltpu.SemaphoreType.DMA,
            pltpu.SemaphoreType.DMA,
            pltpu.SemaphoreType.DMA,
            pltpu.SemaphoreType.DMA,
        ],
    )
    return f(featsF, nl2, rowid_flat)


@jax.jit
def kernel(feats, cu_seqlens, layer_ids, fc1_w, fc1_b, fc2_w, fc2_b,
           attn_w, attn_b, fcc_w, fcc_b):
    fccA = fcc_w[:, :D]
    fccB = fcc_w[:, D:]
    lid_mat = layer_ids.astype(jnp.int32).reshape(R_MAT, D)

    featsF, w_mat, rowid_mat = _run_k1(
        feats, lid_mat, cu_seqlens.astype(jnp.int32), attn_w, attn_b,
        fccB, fcc_b.reshape(1, D))

    rowid_flat = rowid_mat.reshape(N_TOK)
    w_flat = w_mat.reshape(N_TOK)

    featsGv = feats.reshape(N_TOK // 8, 8, D // 16, 16).transpose(
        2, 0, 1, 3).reshape(D // 16, N_TOK // 8, D)
    avgG, maxG, attG = _run_k2(featsGv, rowid_flat, w_flat)

    def _untile(x):
        return x.reshape(8, N_SEG, 16).transpose(1, 0, 2).reshape(N_SEG, D)

    avgP = _untile(avgG)
    maxP = _untile(maxG)
    attP = _untile(attG)
    nl2 = _run_k3(avgP, maxP, attP, fc1_w, fc1_b, fc2_w, fc2_b, fccA)
    return _run_k4(featsF, nl2, rowid_flat)


# speedup vs baseline: 1.5129x; 1.5129x over previous
import math

import jax
import jax.numpy as jnp
from jax import lax
from jax.experimental import pallas as pl
from jax.experimental.pallas import tpu as pltpu
from jax.experimental.pallas import tpu_sc as plsc

N_TOK = 32768
D = 128
N_BATCH = 16
N_SEG = 1024
NC = 2
NS = 16
NW = NC * NS
ROWS_PER_TILE = N_SEG // NW
TOK_PER_TILE = N_TOK // NW

R_MAT = 256
SCAN_CHUNK = 2048
GATHER_SUB = 80
N_GBUF = 4
K4_SUB = 128

_LIST_CAP = N_TOK + GATHER_SUB

_SC_PARAMS = pltpu.CompilerParams(needs_layout_passes=False)

K1_BLOCKS = 8
K1_R = R_MAT // K1_BLOCKS
K1_TOK = N_TOK // K1_BLOCKS


def _k1a_body(feats3_ref, lid_ref, cu_ref, attw_ref, attb_ref, fccB_ref,
              fccb_ref, featsF_ref, logits_ref, rowid_ref):
    g = pl.program_id(0)
    f3 = feats3_ref[...]
    aw = attw_ref[...].reshape(1, 1, D)
    logits_ref[...] = jnp.sum(f3 * aw, axis=2) + attb_ref[0]

    i0 = lax.broadcasted_iota(jnp.int32, (K1_R, D), 0)
    i1 = lax.broadcasted_iota(jnp.int32, (K1_R, D), 1)
    idx = g * K1_TOK + i0 * D + i1
    b = jnp.zeros((K1_R, D), jnp.int32)
    for j in range(1, N_BATCH):
        b = b + (idx >= cu_ref[j]).astype(jnp.int32)

    lid = lid_ref[...]
    rowid_ref[...] = (lid & 31) * 32 + b * 2 + (lid >> 5)

    feats = f3.reshape(K1_TOK, D)
    featsF_ref[...] = (
        lax.dot_general(feats, fccB_ref[...], (((1,), (1,)), ((), ())),
                        preferred_element_type=jnp.float32)
        + fccb_ref[...]
    )


def _k1b_body(logits_ref, cu_ref, w_ref):
    logits = logits_ref[...]
    i0 = lax.broadcasted_iota(jnp.int32, (R_MAT, D), 0)
    i1 = lax.broadcasted_iota(jnp.int32, (R_MAT, D), 1)
    idx = i0 * D + i1
    b = jnp.zeros((R_MAT, D), jnp.int32)
    for j in range(1, N_BATCH):
        b = b + (idx >= cu_ref[j]).astype(jnp.int32)

    m = jnp.max(logits)
    e = jnp.exp(logits - m)
    denom = jnp.ones((R_MAT, D), jnp.float32)
    for j in range(N_BATCH):
        mask = b == j
        zj = jnp.sum(jnp.where(mask, e, 0.0))
        denom = jnp.where(mask, zj, denom)
    w_ref[...] = e / denom


def _run_k1(feats, lid_mat, cu_seqlens, attn_w, attn_b, fccB, fcc_b):
    feats3 = feats.reshape(R_MAT, D, D)
    featsF, logits, rowid = pl.pallas_call(
        _k1a_body,
        grid=(K1_BLOCKS,),
        out_shape=[
            jax.ShapeDtypeStruct((N_TOK, D), jnp.float32),
            jax.ShapeDtypeStruct((R_MAT, D), jnp.float32),
            jax.ShapeDtypeStruct((R_MAT, D), jnp.int32),
        ],
        in_specs=[
            pl.BlockSpec((K1_R, D, D), lambda i: (i, 0, 0)),
            pl.BlockSpec((K1_R, D), lambda i: (i, 0)),
            pl.BlockSpec(memory_space=pltpu.SMEM),
            pl.BlockSpec((1, D), lambda i: (0, 0)),
            pl.BlockSpec(memory_space=pltpu.SMEM),
            pl.BlockSpec((D, D), lambda i: (0, 0)),
            pl.BlockSpec((1, D), lambda i: (0, 0)),
        ],
        out_specs=[
            pl.BlockSpec((K1_TOK, D), lambda i: (i, 0)),
            pl.BlockSpec((K1_R, D), lambda i: (i, 0)),
            pl.BlockSpec((K1_R, D), lambda i: (i, 0)),
        ],
    )(feats3, lid_mat, cu_seqlens, attn_w, attn_b, fccB, fcc_b)

    w = pl.pallas_call(
        _k1b_body,
        out_shape=jax.ShapeDtypeStruct((R_MAT, D), jnp.float32),
        in_specs=[
            pl.BlockSpec(memory_space=pltpu.VMEM),
            pl.BlockSpec(memory_space=pltpu.SMEM),
        ],
    )(logits, cu_seqlens)
    return featsF, w, rowid


def _k2_body(feats_hbm, rid_hbm, w_hbm, avg_hbm, max_hbm, att_hbm,
             chunk0, chunk1, wchunk0, wchunk1, rid_l, w_l,
             idxb0, idxb1, idxb2, idxb3, rows0, rows1, rows2, rows3,
             acc_s, acc_m, acc_a, cnt_v,
             csem0, csem1, gsem0, gsem1, gsem2, gsem3):
    wid = lax.axis_index("s") * NC + lax.axis_index("c")

    chunks = (chunk0, chunk1)
    wchunks = (wchunk0, wchunk1)
    csems = (csem0, csem1)
    rows = (rows0, rows1, rows2, rows3)
    idxb = (idxb0, idxb1, idxb2, idxb3)
    gsems = (gsem0, gsem1, gsem2, gsem3)

    zeros16 = jnp.zeros((16,), jnp.float32)
    neglarge = jnp.full((16,), -3.4e38, jnp.float32)

    def init_fn(r, _):
        for d in range(D // 16):
            sl = pl.ds(d * 16, 16)
            acc_s[r, sl] = zeros16
            acc_m[r, sl] = neglarge
            acc_a[r, sl] = zeros16
        cnt_v[r] = 0
        return 0

    lax.fori_loop(0, ROWS_PER_TILE + 1, init_fn, 0)

    n_chunks = N_TOK // SCAN_CHUNK

    def chunk_src(c):
        return rid_hbm.at[pl.ds(c * SCAN_CHUNK, SCAN_CHUNK)]

    def wchunk_src(c):
        return w_hbm.at[pl.ds(c * SCAN_CHUNK, SCAN_CHUNK)]

    pltpu.async_copy(chunk_src(0), chunks[0], csems[0])
    pltpu.async_copy(wchunk_src(0), wchunks[0], csems[0])
    cur = 0
    for c in range(n_chunks):
        p = c & 1
        if c + 1 < n_chunks:
            pltpu.async_copy(chunk_src(c + 1), chunks[1 - p], csems[1 - p])
            pltpu.async_copy(wchunk_src(c + 1), wchunks[1 - p], csems[1 - p])
        pltpu.make_async_copy(chunk_src(c), chunks[p], csems[p]).wait()
        pltpu.make_async_copy(wchunk_src(c), wchunks[p], csems[p]).wait()
        cbuf = chunks[p]
        wcbuf = wchunks[p]

        def vec_fn(v, cur, c=c, cbuf=cbuf, wcbuf=wcbuf):
            rv = cbuf[pl.ds(v * 16, 16)]
            wv = wcbuf[pl.ds(v * 16, 16)]
            own = lax.shift_right_logical(rv, 5) == wid
            tok = c * SCAN_CHUNK + v * 16 + lax.iota(jnp.int32, 16)
            pack = lax.shift_left(rv, 15) | tok
            inc = plsc.cumsum(own.astype(jnp.int32))
            pos = cur + inc - 1
            plsc.store_scatter(rid_l, [pos], pack, mask=own)
            plsc.store_scatter(w_l, [pos], wv, mask=own)
            return cur + plsc.all_reduce_population_count(own)[0]

        cur = lax.fori_loop(0, SCAN_CHUNK // 16, vec_fn, cur, unroll=4)
    n_own = cur

    pad_vec = jnp.zeros((16,), jnp.int32) + ((wid * 32 + 32) << 15)
    for k in range(GATHER_SUB // 16):
        rid_l[pl.ds(n_own + k * 16, 16)] = pad_vec

    nsub = (n_own + GATHER_SUB - 1) // GATHER_SUB

    def g_issue(s, b):
        for k in range(GATHER_SUB // 16):
            pk = rid_l[pl.ds(s * GATHER_SUB + k * 16, 16)]
            idxb[b][pl.ds(k * 16, 16)] = pk & 32767
        pltpu.async_copy(feats_hbm.at[idxb[b]], rows[b], gsems[b])

    def g_wait(s, b):
        pltpu.make_async_copy(feats_hbm.at[idxb[b]], rows[b], gsems[b]).wait()

    def g_process(s, b):
        nin = jnp.maximum(0, jnp.minimum(GATHER_SUB, n_own - s * GATHER_SUB))
        ngrp = lax.shift_right_logical(nin + 15, 4)
        rbuf = rows[b]

        def grp_fn(t, _):
            base = s * GATHER_SUB + t * 16
            pkv = rid_l[pl.ds(base, 16)]
            wtv = w_l[pl.ds(base, 16)]
            for l in range(16):
                loc = lax.shift_right_logical(pkv[l], 15) - wid * 32
                wt = wtv[l]
                i = t * 16 + l
                for d in range(D // 16):
                    sl = pl.ds(d * 16, 16)
                    v = rbuf[i, sl]
                    acc_s[loc, sl] = acc_s[loc, sl] + v
                    acc_m[loc, sl] = jnp.maximum(acc_m[loc, sl], v)
                    acc_a[loc, sl] = acc_a[loc, sl] + v * wt
                cnt_v[loc] = cnt_v[loc] + 1
            return 0

        lax.fori_loop(0, ngrp, grp_fn, 0)

    for j in range(N_GBUF - 1):
        @pl.when(nsub > j)
        def _(j=j):
            g_issue(j, j)

    def quad_fn(q, _):
        s0 = 4 * q
        for j in range(N_GBUF):
            s = s0 + j

            @pl.when(s + (N_GBUF - 1) < nsub)
            def _(s=s, j=j):
                g_issue(s + (N_GBUF - 1), (j + N_GBUF - 1) % N_GBUF)

            @pl.when(s < nsub)
            def _(s=s, j=j):
                g_wait(s, j)

            g_process(s, j)
        return 0

    lax.fori_loop(0, (nsub + N_GBUF - 1) // N_GBUF, quad_fn, 0)

    def fin_fn(r, _):
        c = cnt_v[r]
        cf = jnp.maximum(c, 1).astype(jnp.float32)
        has = c > 0
        for d in range(D // 16):
            sl = pl.ds(d * 16, 16)
            acc_s[r, sl] = acc_s[r, sl] / cf
            acc_m[r, sl] = jnp.where(has, acc_m[r, sl], zeros16)
        return 0

    lax.fori_loop(0, ROWS_PER_TILE, fin_fn, 0)

    base = wid * ROWS_PER_TILE
    pltpu.sync_copy(acc_s.at[pl.ds(0, ROWS_PER_TILE)],
                    avg_hbm.at[pl.ds(base, ROWS_PER_TILE), :])
    pltpu.sync_copy(acc_m.at[pl.ds(0, ROWS_PER_TILE)],
                    max_hbm.at[pl.ds(base, ROWS_PER_TILE), :])
    pltpu.sync_copy(acc_a.at[pl.ds(0, ROWS_PER_TILE)],
                    att_hbm.at[pl.ds(base, ROWS_PER_TILE), :])


def _run_k2(feats, rowid_flat, w_flat):
    mesh = plsc.VectorSubcoreMesh(core_axis_name="c", subcore_axis_name="s")
    f = pl.kernel(
        _k2_body,
        out_type=[
            jax.ShapeDtypeStruct((N_SEG, D), jnp.float32),
            jax.ShapeDtypeStruct((N_SEG, D), jnp.float32),
            jax.ShapeDtypeStruct((N_SEG, D), jnp.float32),
        ],
        mesh=mesh,
        compiler_params=_SC_PARAMS,
        scratch_types=[
            pltpu.VMEM((SCAN_CHUNK,), jnp.int32),
            pltpu.VMEM((SCAN_CHUNK,), jnp.int32),
            pltpu.VMEM((SCAN_CHUNK,), jnp.float32),
            pltpu.VMEM((SCAN_CHUNK,), jnp.float32),
            pltpu.VMEM((_LIST_CAP,), jnp.int32),
            pltpu.VMEM((_LIST_CAP,), jnp.float32),
            pltpu.VMEM((GATHER_SUB,), jnp.int32),
            pltpu.VMEM((GATHER_SUB,), jnp.int32),
            pltpu.VMEM((GATHER_SUB,), jnp.int32),
            pltpu.VMEM((GATHER_SUB,), jnp.int32),
            pltpu.VMEM((GATHER_SUB, D), jnp.float32),
            pltpu.VMEM((GATHER_SUB, D), jnp.float32),
            pltpu.VMEM((GATHER_SUB, D), jnp.float32),
            pltpu.VMEM((GATHER_SUB, D), jnp.float32),
            pltpu.VMEM((ROWS_PER_TILE + 1, D), jnp.float32),
            pltpu.VMEM((ROWS_PER_TILE + 1, D), jnp.float32),
            pltpu.VMEM((ROWS_PER_TILE + 1, D), jnp.float32),
            pltpu.SMEM((ROWS_PER_TILE + 1,), jnp.int32),
            pltpu.SemaphoreType.DMA,
            pltpu.SemaphoreType.DMA,
            pltpu.SemaphoreType.DMA,
            pltpu.SemaphoreType.DMA,
            pltpu.SemaphoreType.DMA,
            pltpu.SemaphoreType.DMA,
        ],
    )
    return f(feats, rowid_flat, w_flat)


def _k3_body(avg_ref, max_ref, att_ref, w1_ref, b1_ref, w2_ref, b2_ref,
             fccA_ref, nl2_ref):
    fcat = jnp.concatenate([avg_ref[...], max_ref[...], att_ref[...]], axis=1)
    h = lax.dot_general(fcat, w1_ref[...], (((1,), (1,)), ((), ())),
                        preferred_element_type=jnp.float32) + b1_ref[...]
    h = 0.5 * h * (1.0 + lax.erf(h * (1.0 / math.sqrt(2.0))))
    nl = lax.dot_general(h, w2_ref[...], (((1,), (1,)), ((), ())),
                         preferred_element_type=jnp.float32) + b2_ref[...]
    nl2_ref[...] = lax.dot_general(nl, fccA_ref[...], (((1,), (1,)), ((), ())),
                                   preferred_element_type=jnp.float32)


def _run_k3(avgP, maxP, attP, fc1_w, fc1_b, fc2_w, fc2_b, fccA):
    return pl.pallas_call(
        _k3_body,
        out_shape=jax.ShapeDtypeStruct((N_SEG, D), jnp.float32),
    )(avgP, maxP, attP, fc1_w, fc1_b.reshape(1, -1), fc2_w,
      fc2_b.reshape(1, -1), fccA)


def _k4_body(featsF_hbm, nl2_hbm, rid_hbm, out_hbm,
             ridx0, ridx1, rows0, rows1, fbuf0, fbuf1,
             sem0, sem1, osem0, osem1):
    wid = lax.axis_index("s") * NC + lax.axis_index("c")
    base = wid * TOK_PER_TILE
    n_sub = TOK_PER_TILE // K4_SUB

    ridx = (ridx0, ridx1)
    rows = (rows0, rows1)
    fbuf = (fbuf0, fbuf1)
    sems = (sem0, sem1)
    osems = (osem0, osem1)

    def issue(s):
        p = s & 1
        t0 = base + s * K4_SUB
        pltpu.sync_copy(rid_hbm.at[pl.ds(t0, K4_SUB)], ridx[p])
        pltpu.async_copy(nl2_hbm.at[ridx[p]], rows[p], sems[p])
        pltpu.async_copy(featsF_hbm.at[pl.ds(t0, K4_SUB), :], fbuf[p], sems[p])

    def wait_in(s):
        p = s & 1
        t0 = base + s * K4_SUB
        pltpu.make_async_copy(nl2_hbm.at[ridx[p]], rows[p], sems[p]).wait()
        pltpu.make_async_copy(featsF_hbm.at[pl.ds(t0, K4_SUB), :], fbuf[p],
                              sems[p]).wait()

    def wait_out(s):
        p = s & 1
        t0 = base + s * K4_SUB
        pltpu.make_async_copy(fbuf[p], out_hbm.at[pl.ds(t0, K4_SUB), :],
                              osems[p]).wait()

    issue(0)
    for s in range(n_sub):
        p = s & 1
        t0 = base + s * K4_SUB
        if s + 1 < n_sub:
            if s >= 1:
                wait_out(s - 1)
            issue(s + 1)
        wait_in(s)

        def row_fn(r, _, p=p):
            for d in range(D // 16):
                sl = pl.ds(d * 16, 16)
                fbuf[p][r, sl] = fbuf[p][r, sl] + rows[p][r, sl]
            return 0

        lax.fori_loop(0, K4_SUB, row_fn, 0)
        pltpu.async_copy(fbuf[p], out_hbm.at[pl.ds(t0, K4_SUB), :], osems[p])
    if n_sub >= 2:
        wait_out(n_sub - 2)
    wait_out(n_sub - 1)


def _run_k4(featsF, nl2, rowid_flat):
    mesh = plsc.VectorSubcoreMesh(core_axis_name="c", subcore_axis_name="s")
    f = pl.kernel(
        _k4_body,
        out_type=jax.ShapeDtypeStruct((N_TOK, D), jnp.float32),
        mesh=mesh,
        compiler_params=_SC_PARAMS,
        scratch_types=[
            pltpu.VMEM((K4_SUB,), jnp.int32),
            pltpu.VMEM((K4_SUB,), jnp.int32),
            pltpu.VMEM((K4_SUB, D), jnp.float32),
            pltpu.VMEM((K4_SUB, D), jnp.float32),
            pltpu.VMEM((K4_SUB, D), jnp.float32),
            pltpu.VMEM((K4_SUB, D), jnp.float32),
            pltpu.SemaphoreType.DMA,
            pltpu.SemaphoreType.DMA,
            pltpu.SemaphoreType.DMA,
            pltpu.SemaphoreType.DMA,
        ],
    )
    return f(featsF, nl2, rowid_flat)


@jax.jit
def kernel(feats, cu_seqlens, layer_ids, fc1_w, fc1_b, fc2_w, fc2_b,
           attn_w, attn_b, fcc_w, fcc_b):
    fccA = fcc_w[:, :D]
    fccB = fcc_w[:, D:]
    lid_mat = layer_ids.astype(jnp.int32).reshape(R_MAT, D)

    featsF, w_mat, rowid_mat = _run_k1(
        feats, lid_mat, cu_seqlens.astype(jnp.int32), attn_w, attn_b,
        fccB, fcc_b.reshape(1, D))

    rowid_flat = rowid_mat.reshape(N_TOK)
    w_flat = w_mat.reshape(N_TOK)

    avgP, maxP, attP = _run_k2(feats, rowid_flat, w_flat)
    nl2 = _run_k3(avgP, maxP, attP, fc1_w, fc1_b, fc2_w, fc2_b, fccA)
    return _run_k4(featsF, nl2, rowid_flat)
